# baseline (speedup 1.0000x reference)
import functools

import jax
import jax.numpy as jnp
import numpy as np
from jax import lax
from jax.experimental import pallas as pl
from jax.experimental.pallas import tpu as pltpu, tpu_sc as plsc

N = 10000
E = 320000
D = 128
HEADS = 8
HD = D // HEADS
RBF = 20
N1 = 64
N2 = 32
VDIM = N1 * 3 + N2 * 5
NUM_GRAPHS = 64

NC = 2
NS = 16
NW = NC * NS
CH = 128

KE = 79
EP = NW * KE * CH
EB = 512
NEB = EP // EB
NB = 512

KA = 3
AP = NW * KA * CH
NPAD = 10112
NFOLD = 640


def _mesh():
    return plsc.VectorSubcoreMesh(
        core_axis_name="c", subcore_axis_name="s", num_cores=NC, num_subcores=NS)


@functools.partial(jax.jit, static_argnames=("k_chunks", "row_w"))
def _sc_gather(table, idx3, *, k_chunks, row_w):
    ep = NW * k_chunks * CH

    @functools.partial(
        pl.kernel, mesh=_mesh(),
        out_type=jax.ShapeDtypeStruct((ep, row_w), jnp.float32),
        scratch_types=[
            pltpu.VMEM((k_chunks, CH), jnp.int32),
            pltpu.VMEM((CH, row_w), jnp.float32),
            pltpu.SemaphoreType.DMA,
        ],
    )
    def k(table_hbm, idx_hbm, out_hbm, idx_v, buf_v, sem):
        c = lax.axis_index("c")
        s = lax.axis_index("s")
        wid = s * NC + c
        pltpu.sync_copy(idx_hbm.at[wid], idx_v)
        base = wid * k_chunks * CH

        def body(j, carry):
            pltpu.async_copy(table_hbm.at[idx_v.at[j]], buf_v, sem).wait()
            pltpu.sync_copy(buf_v, out_hbm.at[pl.ds(base + j * CH, CH)])
            return carry

        lax.fori_loop(0, k_chunks, body, 0)

    return k(table, idx3)


@functools.partial(jax.jit, static_argnames=("k_chunks", "row_w", "n_rows"))
def _sc_scatter_add(vals, idx3, *, k_chunks, row_w, n_rows):
    rps = n_rows // NS
    zsz = []
    left = rps
    while left > 0:
        zsz.append(min(CH, left))
        left -= zsz[-1]
    zeros = jnp.zeros((CH, row_w), jnp.float32)

    @functools.partial(
        pl.kernel, mesh=_mesh(),
        out_type=jax.ShapeDtypeStruct((NC, n_rows, row_w), jnp.float32),
        scratch_types=[
            pltpu.VMEM((k_chunks, CH), jnp.int32),
            pltpu.VMEM((CH, row_w), jnp.float32),
            pltpu.VMEM_SHARED((n_rows, row_w), jnp.float32),
            pltpu.SemaphoreType.DMA,
        ],
    )
    def k(vals_hbm, idx_hbm, zeros_hbm, out_hbm, idx_v, buf_v, acc, sem):
        c = lax.axis_index("c")
        s = lax.axis_index("s")
        wid = s * NC + c
        off = s * rps
        rel = 0
        for sz in zsz:
            pltpu.sync_copy(zeros_hbm.at[pl.ds(0, sz)], acc.at[pl.ds(off + rel, sz)])
            rel += sz
        plsc.subcore_barrier()
        pltpu.sync_copy(idx_hbm.at[wid], idx_v)
        base = wid * k_chunks * CH

        def body(j, carry):
            pltpu.sync_copy(vals_hbm.at[pl.ds(base + j * CH, CH)], buf_v)
            pltpu.sync_copy(buf_v, acc.at[idx_v.at[j]], add=True)
            return carry

        lax.fori_loop(0, k_chunks, body, 0)
        plsc.subcore_barrier()
        pltpu.sync_copy(acc.at[pl.ds(off, rps)], out_hbm.at[c, pl.ds(off, rps)])

    return k(vals, idx3, zeros)


def _edge_spec(w, col=0):
    return pl.BlockSpec((EB, w), lambda i, c=col: (i, c))


def _full(shape):
    nd = len(shape)
    return pl.BlockSpec(shape, lambda i: (0,) * nd)


def _tc_project(xs, Wcat):
    grid = (pl.cdiv(N, NB),)

    def body(x_ref, w_ref, q_ref, kv_ref, r_ref):
        h = jnp.dot(x_ref[...], w_ref[...], preferred_element_type=jnp.float32)
        q_ref[...] = h[:, :D]
        kv_ref[...] = h[:, D:3 * D]
        r_ref[...] = h[:, 3 * D:]

    return pl.pallas_call(
        body,
        grid=grid,
        in_specs=[pl.BlockSpec((NB, D), lambda i: (i, 0)), _full((D, 4 * D))],
        out_specs=[pl.BlockSpec((NB, D), lambda i: (i, 0)),
                   pl.BlockSpec((NB, 2 * D), lambda i: (i, 0)),
                   pl.BlockSpec((NB, D), lambda i: (i, 0))],
        out_shape=[jax.ShapeDtypeStruct((N, D), jnp.float32),
                   jax.ShapeDtypeStruct((N, 2 * D), jnp.float32),
                   jax.ShapeDtypeStruct((N, D), jnp.float32)],
    )(xs, Wcat)


def _tc_logits(qg, kvg, rbf_p, mask_p, dm16, ddiv, Wrbf, SEL, R1c, R2c):
    def body(q_ref, k_ref, rbf_ref, m_ref, dm_ref, dv_ref, w_ref, sel_ref,
             r1_ref, r2_ref, out_ref, den_ref):
        i = pl.program_id(0)
        re = jnp.dot(rbf_ref[...], w_ref[...], preferred_element_type=jnp.float32)
        s = q_ref[...] * k_ref[...] * re
        lg = jnp.dot(s, sel_ref[...], preferred_element_type=jnp.float32) * (1.0 / np.sqrt(HD))
        ex = jnp.exp(jnp.minimum(lg, 60.0)) * m_ref[...]
        out_ref[...] = jnp.concatenate([ex, jnp.zeros_like(ex)], axis=1)
        placed = jnp.dot(dm_ref[...], r1_ref[...], preferred_element_type=jnp.float32) \
            * jnp.dot(ex, r2_ref[...], preferred_element_type=jnp.float32)
        oh = (dv_ref[...] == lax.broadcasted_iota(jnp.int32, (EB, NFOLD), 1).astype(jnp.float32))
        contrib = lax.dot_general(oh.astype(jnp.bfloat16),
                                  placed.astype(jnp.bfloat16),
                                  (((0,), (0,)), ((), ())),
                                  preferred_element_type=jnp.float32)

        @pl.when(i == 0)
        def _():
            den_ref[...] = contrib

        @pl.when(i > 0)
        def _():
            den_ref[...] += contrib

    return pl.pallas_call(
        body,
        grid=(NEB,),
        in_specs=[_edge_spec(D), _edge_spec(D, 0), _edge_spec(RBF), _edge_spec(1),
                  _edge_spec(16), _edge_spec(1),
                  _full((RBF, D)), _full((D, HEADS)), _full((16, D)),
                  _full((HEADS, D))],
        out_specs=[_edge_spec(16), pl.BlockSpec((NFOLD, D), lambda i: (0, 0))],
        out_shape=[jax.ShapeDtypeStruct((EP, 16), jnp.float32),
                   jax.ShapeDtypeStruct((NFOLD, D), jnp.float32)],
    )(qg, kvg, rbf_p, mask_p, dm16, ddiv, Wrbf, SEL, R1c, R2c)


def _tc_messages(ex, dtab, dm16, ddiv, kvg, rsh_p, env_p, Wg12, EXP8,
                 A1, B1, A2, B2, R1c, SS8):
    def body(ex_ref, dt_ref, dm_ref, dv_ref, v_ref, rsh_ref, env_ref,
             wg_ref, e8_ref, a1_ref, b1_ref, a2_ref, b2_ref, r1_ref, ss_ref,
             ms_ref, va_ref, vb_ref, vc_ref):
        oh = (dv_ref[...] == lax.broadcasted_iota(jnp.int32, (EB, NFOLD), 1).astype(jnp.float32))
        rowsel = jnp.dot(oh.astype(jnp.bfloat16), dt_ref[...].astype(jnp.bfloat16),
                         preferred_element_type=jnp.float32)
        msk = jnp.dot(dm_ref[...], r1_ref[...], preferred_element_type=jnp.float32)
        dsum = jnp.dot(rowsel * msk, ss_ref[...], preferred_element_type=jnp.float32)
        attn = ex_ref[:, :HEADS] / (dsum + 1e-16) * env_ref[...]
        ms = jnp.dot(attn, e8_ref[...], preferred_element_type=jnp.float32) * v_ref[...]
        g = jnp.dot(ms, wg_ref[...], preferred_element_type=jnp.float32)
        rsh = rsh_ref[...]
        v1 = jnp.dot(g[:, :N1], a1_ref[...], preferred_element_type=jnp.float32) \
            * jnp.dot(rsh[:, 0:3], b1_ref[...], preferred_element_type=jnp.float32)
        v2 = jnp.dot(g[:, N1:], a2_ref[...], preferred_element_type=jnp.float32) \
            * jnp.dot(rsh[:, 3:8], b2_ref[...], preferred_element_type=jnp.float32)
        ms_ref[...] = ms
        va_ref[...] = v1[:, :D]
        vb_ref[...] = jnp.concatenate([v1[:, D:], v2[:, :2 * D - N1 * 3]], axis=1)
        vc_ref[...] = jnp.concatenate(
            [v2[:, 2 * D - N1 * 3:],
             jnp.zeros((EB, 3 * D - N1 * 3 - N2 * 5), jnp.float32)], axis=1)

    return pl.pallas_call(
        body,
        grid=(NEB,),
        in_specs=[_edge_spec(16), _full((NFOLD, D)), _edge_spec(16), _edge_spec(1),
                  _edge_spec(D, 1), _edge_spec(8),
                  _edge_spec(1), _full((D, N1 + N2)), _full((HEADS, D)),
                  _full((N1, N1 * 3)), _full((3, N1 * 3)),
                  _full((N2, N2 * 5)), _full((5, N2 * 5)),
                  _full((16, D)), _full((D, HEADS))],
        out_specs=[_edge_spec(D), _edge_spec(D), _edge_spec(D), _edge_spec(D)],
        out_shape=[jax.ShapeDtypeStruct((EP, D), jnp.float32),
                   jax.ShapeDtypeStruct((EP, D), jnp.float32),
                   jax.ShapeDtypeStruct((EP, D), jnp.float32),
                   jax.ShapeDtypeStruct((EP, D), jnp.float32)],
    )(ex, dtab, dm16, ddiv, kvg, rsh_p, env_p, Wg12, EXP8, A1, B1, A2, B2, R1c, SS8)


def _tc_update(xs, xv, ms2, va2, vb2, vc2, root, consts):
    M1, M2, S3, S5, G1, G2, Wu1, bu1, Wu2, bu2, ln_g, ln_b, eq3, eq5 = consts

    def body(xs_ref, xv_ref, ms_ref, va_ref, vb_ref, vc_ref, rt_ref,
             m1_ref, m2_ref, s3_ref, s5_ref, g1_ref, g2_ref,
             wu1_ref, bu1_ref, wu2_ref, bu2_ref, lg_ref, lb_ref, e3_ref, e5_ref,
             xso_ref, xvo_ref):
        xs_in = xs_ref[...]
        xv_in = xv_ref[...]
        ms = ms_ref[...]
        va = va_ref[0] + va_ref[1]
        vb = vb_ref[0] + vb_ref[1]
        vc = vc_ref[0] + vc_ref[1]
        a1 = jnp.concatenate([va, vb[:, :N1 * 3 - D]], axis=1)
        a2 = jnp.concatenate([vb[:, N1 * 3 - D:], vc[:, :N2 * 5 - 2 * D + N1 * 3]], axis=1)
        new_s = ms[0] + ms[1] + rt_ref[...]
        xv1 = xv_in[:, :N1 * 3]
        xv2 = xv_in[:, N1 * 3:]
        nv1 = a1 + jnp.dot(xv1, m1_ref[...], preferred_element_type=jnp.float32)
        nv2 = a2 + jnp.dot(xv2, m2_ref[...], preferred_element_type=jnp.float32)
        n1 = jnp.sqrt(jnp.dot(nv1 * nv1, s3_ref[...], preferred_element_type=jnp.float32) + 1e-8)
        n2 = jnp.sqrt(jnp.dot(nv2 * nv2, s5_ref[...], preferred_element_type=jnp.float32) + 1e-8)
        h = jnp.concatenate([new_s, n1, n2], axis=1)
        h = jnp.dot(h, wu1_ref[...], preferred_element_type=jnp.float32) + bu1_ref[...]
        h = h * jax.nn.sigmoid(h)
        h = jnp.dot(h, wu2_ref[...], preferred_element_type=jnp.float32) + bu2_ref[...]
        s_out = h[:, :D]
        gates = jax.nn.sigmoid(h[:, D:])
        v1g = nv1 * jnp.dot(gates[:, :N1], g1_ref[...], preferred_element_type=jnp.float32)
        v2g = nv2 * jnp.dot(gates[:, N1:], g2_ref[...], preferred_element_type=jnp.float32)
        mu = jnp.mean(s_out, axis=-1, keepdims=True)
        dvi = s_out - mu
        var = jnp.mean(dvi * dvi, axis=-1, keepdims=True)
        xs_ln = dvi / jnp.sqrt(var + 1e-8) * lg_ref[...] + lb_ref[...]
        msq = (jnp.sum(v1g * v1g, axis=-1, keepdims=True)
               + jnp.sum(v2g * v2g, axis=-1, keepdims=True)) / (N1 + N2)
        inv = 1.0 / jnp.sqrt(msq + 1e-8)
        xv_ln = jnp.concatenate([v1g * inv * e3_ref[...], v2g * inv * e5_ref[...]], axis=1)
        xso_ref[...] = xs_ln + xs_in
        xvo_ref[...] = xv_ln + xv_in

    U = D + N1 + N2
    return pl.pallas_call(
        body,
        grid=(pl.cdiv(N, NB),),
        in_specs=[pl.BlockSpec((NB, D), lambda i: (i, 0)),
                  pl.BlockSpec((NB, VDIM), lambda i: (i, 0)),
                  pl.BlockSpec((NC, NB, D), lambda i: (0, i, 0)),
                  pl.BlockSpec((NC, NB, D), lambda i: (0, i, 0)),
                  pl.BlockSpec((NC, NB, D), lambda i: (0, i, 0)),
                  pl.BlockSpec((NC, NB, D), lambda i: (0, i, 0)),
                  pl.BlockSpec((NB, D), lambda i: (i, 0)),
                  _full((N1 * 3, N1 * 3)), _full((N2 * 5, N2 * 5)),
                  _full((N1 * 3, N1)), _full((N2 * 5, N2)),
                  _full((N1, N1 * 3)), _full((N2, N2 * 5)),
                  _full((U, 2 * U)), _full((1, 2 * U)), _full((2 * U, U)), _full((1, U)),
                  _full((1, D)), _full((1, D)), _full((1, N1 * 3)), _full((1, N2 * 5))],
        out_specs=[pl.BlockSpec((NB, D), lambda i: (i, 0)),
                   pl.BlockSpec((NB, VDIM), lambda i: (i, 0))],
        out_shape=[jax.ShapeDtypeStruct((N, D), jnp.float32),
                   jax.ShapeDtypeStruct((N, VDIM), jnp.float32)],
    )(xs, xv, ms2, va2, vb2, vc2, root,
      M1, M2, S3, S5, G1, G2, Wu1, bu1, Wu2, bu2, ln_g, ln_b, eq3, eq5)


def _tc_renv(rbf_p, env_p):
    def body(rbf_ref, env_ref, out_ref):
        re = rbf_ref[...] * env_ref[...]
        out_ref[...] = jnp.concatenate(
            [re, jnp.zeros((EB, D - RBF), jnp.float32)], axis=1)

    return pl.pallas_call(
        body,
        grid=(NEB,),
        in_specs=[_edge_spec(RBF), _edge_spec(1)],
        out_specs=_edge_spec(D),
        out_shape=jax.ShapeDtypeStruct((EP, D), jnp.float32),
    )(rbf_p, env_p)


def _tc_readout(xs, ne2, Wr, mlp, Woutp, boutp):
    (W1, b1), (W2, b2), (W3, b3) = mlp

    def body(xs_ref, ne_ref, wr_ref, w1_ref, b1_ref, w2_ref, b2_ref,
             w3_ref, b3_ref, wo_ref, bo_ref, out_ref):
        ne = ne_ref[...]
        node20 = ne[0, :, :RBF] + ne[1, :, :RBF]
        h = xs_ref[...] + jnp.dot(node20, wr_ref[...], preferred_element_type=jnp.float32)
        for w_ref, bias_ref in ((w1_ref, b1_ref), (w2_ref, b2_ref), (w3_ref, b3_ref)):
            h = jnp.dot(h, w_ref[...], preferred_element_type=jnp.float32) + bias_ref[...]
            h = h * jax.nn.sigmoid(h)
        out_ref[...] = jnp.dot(h, wo_ref[...], preferred_element_type=jnp.float32) + bo_ref[...]

    return pl.pallas_call(
        body,
        grid=(pl.cdiv(N, NB),),
        in_specs=[pl.BlockSpec((NB, D), lambda i: (i, 0)),
                  pl.BlockSpec((NC, NB, D), lambda i: (0, i, 0)),
                  _full((RBF, D)), _full((D, D)), _full((1, D)), _full((D, D)),
                  _full((1, D)), _full((D, D)), _full((1, D)), _full((D, 8)), _full((1, 8))],
        out_specs=pl.BlockSpec((NB, 8), lambda i: (i, 0)),
        out_shape=jax.ShapeDtypeStruct((N, 8), jnp.float32),
    )(xs, ne2, Wr, W1, b1, W2, b2, W3, b3, Woutp, boutp)


def _np_consts():
    SEL = np.zeros((D, HEADS), np.float32)
    EXP8 = np.zeros((HEADS, D), np.float32)
    for h in range(HEADS):
        SEL[h * HD:(h + 1) * HD, h] = 1.0
        EXP8[h, h * HD:(h + 1) * HD] = 1.0
    A1 = np.zeros((N1, N1 * 3), np.float32)
    B1 = np.zeros((3, N1 * 3), np.float32)
    for cc in range(N1):
        for dd in range(3):
            A1[cc, cc * 3 + dd] = 1.0
            B1[dd, cc * 3 + dd] = 1.0
    A2 = np.zeros((N2, N2 * 5), np.float32)
    B2 = np.zeros((5, N2 * 5), np.float32)
    for cc in range(N2):
        for dd in range(5):
            A2[cc, cc * 5 + dd] = 1.0
            B2[dd, cc * 5 + dd] = 1.0
    R1c = np.zeros((16, D), np.float32)
    R2c = np.zeros((HEADS, D), np.float32)
    SS8 = np.zeros((D, HEADS), np.float32)
    for ss in range(16):
        for h in range(HEADS):
            R1c[ss, ss * 8 + h] = 1.0
            R2c[h, ss * 8 + h] = 1.0
            SS8[ss * 8 + h, h] = 1.0
    return (jnp.asarray(SEL), jnp.asarray(EXP8), jnp.asarray(A1), jnp.asarray(B1),
            jnp.asarray(A2), jnp.asarray(B2), jnp.asarray(R1c), jnp.asarray(R2c),
            jnp.asarray(SS8))


def kernel(x_scalar, x_vector, edge_index, node_rbf, rsh, edge_index_0,
           atom_batch, envelop_para, params):
    SEL, EXP8, A1, B1, A2, B2, R1c, R2c, SS8 = _np_consts()
    pad = EP - E
    src = edge_index[0]
    dst = edge_index[1]
    dstp = jnp.pad(dst, (0, pad)).reshape(NW, KE, CH)
    srcp = jnp.pad(src, (0, pad)).reshape(NW, KE, CH)
    ei0p = jnp.pad(edge_index_0[1], (0, pad)).reshape(NW, KE, CH)
    abp = jnp.pad(atom_batch, (0, AP - N)).reshape(NW, KA, CH)
    mask_p = (jnp.arange(EP, dtype=jnp.int32) < E).astype(jnp.float32)[:, None]
    rbf_p = jnp.pad(node_rbf, ((0, pad), (0, 0)))
    dstf = jnp.pad(dst, (0, pad))
    dm16 = (jnp.mod(dstf, 16)[:, None] == jnp.arange(16)[None, :]).astype(jnp.float32)
    ddiv = (dstf // 16).astype(jnp.float32)[:, None]
    rsh_p = jnp.pad(rsh, ((0, pad), (0, 0)))
    env_p = jnp.pad(envelop_para, (0, pad))[:, None]

    xs, xv = x_scalar, x_vector
    for p in params['layers']:
        Wcat = jnp.concatenate([p['Wq'], p['Wk'], p['Wv'], p['Wroot']], axis=1)
        Wg12 = jnp.concatenate([p['Wg1'], p['Wg2']], axis=1)
        M1 = jnp.kron(p['Wv1'], jnp.eye(3, dtype=jnp.float32))
        M2 = jnp.kron(p['Wv2'], jnp.eye(5, dtype=jnp.float32))
        eq3 = jnp.repeat(p['eq_g'][:N1], 3)[None, :]
        eq5 = jnp.repeat(p['eq_g'][N1:], 5)[None, :]
        consts = (M1, M2, A1.T, A2.T, A1, A2,
                  p['Wu1'], p['bu1'][None, :], p['Wu2'], p['bu2'][None, :],
                  p['ln_g'][None, :], p['ln_b'][None, :], eq3, eq5)

        q, kv, root = _tc_project(xs, Wcat)
        qg = _sc_gather(q, dstp, k_chunks=KE, row_w=D)
        kvg = _sc_gather(kv, srcp, k_chunks=KE, row_w=2 * D)
        ex, dtab = _tc_logits(qg, kvg, rbf_p, mask_p, dm16, ddiv,
                              p['Wrbf'], SEL, R1c, R2c)
        ms, vma, vmb, vmc = _tc_messages(ex, dtab, dm16, ddiv, kvg, rsh_p,
                                         env_p, Wg12, EXP8, A1, B1, A2, B2,
                                         R1c, SS8)
        ms2 = _sc_scatter_add(ms, dstp, k_chunks=KE, row_w=D, n_rows=NPAD)
        va2 = _sc_scatter_add(vma, dstp, k_chunks=KE, row_w=D, n_rows=NPAD)
        vb2 = _sc_scatter_add(vmb, dstp, k_chunks=KE, row_w=D, n_rows=NPAD)
        vc2 = _sc_scatter_add(vmc, dstp, k_chunks=KE, row_w=D, n_rows=NPAD)
        xs, xv = _tc_update(xs, xv, ms2, va2, vb2, vc2, root, consts)

    ro = params['readout']
    renv = _tc_renv(rbf_p, env_p)
    ne2 = _sc_scatter_add(renv, ei0p, k_chunks=KE, row_w=D, n_rows=NPAD)
    mlp = [(w, b[None, :]) for (w, b) in ro['mlp']]
    Woutp = jnp.pad(ro['Wout'], ((0, 0), (0, 7)))
    boutp = jnp.pad(ro['bout'], (0, 7))[None, :]
    res = _tc_readout(xs, ne2, ro['Wr'], mlp, Woutp, boutp)
    resp = jnp.pad(res, ((0, AP - N), (0, D - 8)))
    out2 = _sc_scatter_add(resp, abp, k_chunks=KA, row_w=D, n_rows=128)
    return (out2[0, :NUM_GRAPHS, 0] + out2[1, :NUM_GRAPHS, 0])


# speedup vs baseline: 1.0016x; 1.0016x over previous
import functools

import jax
import jax.numpy as jnp
import numpy as np
from jax import lax
from jax.experimental import pallas as pl
from jax.experimental.pallas import tpu as pltpu, tpu_sc as plsc

N = 10000
E = 320000
D = 128
HEADS = 8
HD = D // HEADS
RBF = 20
N1 = 64
N2 = 32
VDIM = N1 * 3 + N2 * 5
NUM_GRAPHS = 64

NC = 2
NS = 16
NW = NC * NS
CH = 128

KE = 79
EP = NW * KE * CH
EB = 512
NEB = EP // EB
NB = 512

KA = 3
AP = NW * KA * CH
NPAD = 10112
NFOLD = 640


def _mesh():
    return plsc.VectorSubcoreMesh(
        core_axis_name="c", subcore_axis_name="s", num_cores=NC, num_subcores=NS)


@functools.partial(jax.jit, static_argnames=("k_chunks", "row_w"))
def _sc_gather(table, idx3, *, k_chunks, row_w):
    ep = NW * k_chunks * CH

    @functools.partial(
        pl.kernel, mesh=_mesh(),
        out_type=jax.ShapeDtypeStruct((ep, row_w), jnp.float32),
        scratch_types=[
            pltpu.VMEM((k_chunks, CH), jnp.int32),
            pltpu.VMEM((CH, row_w), jnp.float32),
            pltpu.SemaphoreType.DMA,
        ],
    )
    def k(table_hbm, idx_hbm, out_hbm, idx_v, buf_v, sem):
        c = lax.axis_index("c")
        s = lax.axis_index("s")
        wid = s * NC + c
        pltpu.sync_copy(idx_hbm.at[wid], idx_v)
        base = wid * k_chunks * CH

        def body(j, carry):
            pltpu.async_copy(table_hbm.at[idx_v.at[j]], buf_v, sem).wait()
            pltpu.sync_copy(buf_v, out_hbm.at[pl.ds(base + j * CH, CH)])
            return carry

        lax.fori_loop(0, k_chunks, body, 0)

    return k(table, idx3)


@functools.partial(jax.jit, static_argnames=("k_chunks", "row_w", "n_rows"))
def _sc_scatter_add(vals, idx3, *, k_chunks, row_w, n_rows):
    rps = n_rows // NS
    zsz = []
    left = rps
    while left > 0:
        zsz.append(min(CH, left))
        left -= zsz[-1]
    zeros = jnp.zeros((CH, row_w), jnp.float32)

    @functools.partial(
        pl.kernel, mesh=_mesh(),
        out_type=jax.ShapeDtypeStruct((NC, n_rows, row_w), jnp.float32),
        scratch_types=[
            pltpu.VMEM((k_chunks, CH), jnp.int32),
            pltpu.VMEM((CH, row_w), jnp.float32),
            pltpu.VMEM_SHARED((n_rows, row_w), jnp.float32),
            pltpu.SemaphoreType.DMA,
        ],
    )
    def k(vals_hbm, idx_hbm, zeros_hbm, out_hbm, idx_v, buf_v, acc, sem):
        c = lax.axis_index("c")
        s = lax.axis_index("s")
        wid = s * NC + c
        off = s * rps
        rel = 0
        for sz in zsz:
            pltpu.sync_copy(zeros_hbm.at[pl.ds(0, sz)], acc.at[pl.ds(off + rel, sz)])
            rel += sz
        plsc.subcore_barrier()
        pltpu.sync_copy(idx_hbm.at[wid], idx_v)
        base = wid * k_chunks * CH

        def body(j, carry):
            pltpu.sync_copy(vals_hbm.at[pl.ds(base + j * CH, CH)], buf_v)
            pltpu.sync_copy(buf_v, acc.at[idx_v.at[j]], add=True)
            return carry

        lax.fori_loop(0, k_chunks, body, 0)
        plsc.subcore_barrier()
        pltpu.sync_copy(acc.at[pl.ds(off, rps)], out_hbm.at[c, pl.ds(off, rps)])

    return k(vals, idx3, zeros)


def _edge_spec(w, col=0):
    return pl.BlockSpec((EB, w), lambda i, c=col: (i, c))


def _full(shape):
    nd = len(shape)
    return pl.BlockSpec(shape, lambda i: (0,) * nd)


def _tc_project(xs, Wcat):
    grid = (pl.cdiv(N, NB),)

    def body(x_ref, w_ref, q_ref, kv_ref, r_ref):
        h = jnp.dot(x_ref[...], w_ref[...], preferred_element_type=jnp.float32)
        q_ref[...] = h[:, :D]
        kv_ref[...] = h[:, D:3 * D]
        r_ref[...] = h[:, 3 * D:]

    return pl.pallas_call(
        body,
        grid=grid,
        in_specs=[pl.BlockSpec((NB, D), lambda i: (i, 0)), _full((D, 4 * D))],
        out_specs=[pl.BlockSpec((NB, D), lambda i: (i, 0)),
                   pl.BlockSpec((NB, 2 * D), lambda i: (i, 0)),
                   pl.BlockSpec((NB, D), lambda i: (i, 0))],
        out_shape=[jax.ShapeDtypeStruct((N, D), jnp.float32),
                   jax.ShapeDtypeStruct((N, 2 * D), jnp.float32),
                   jax.ShapeDtypeStruct((N, D), jnp.float32)],
    )(xs, Wcat)


def _tc_logits(qg, kvg, rbf_p, mask_p, dm16, ddiv, Wrbf, SEL, R1c, R2c):
    def body(q_ref, k_ref, rbf_ref, m_ref, dm_ref, dv_ref, w_ref, sel_ref,
             r1_ref, r2_ref, out_ref, den_ref):
        i = pl.program_id(0)
        re = jnp.dot(rbf_ref[...], w_ref[...], preferred_element_type=jnp.float32)
        s = q_ref[...] * k_ref[...] * re
        lg = jnp.dot(s, sel_ref[...], preferred_element_type=jnp.float32) * (1.0 / np.sqrt(HD))
        ex = jnp.exp(jnp.minimum(lg, 60.0)) * m_ref[...]
        out_ref[...] = jnp.concatenate([ex, jnp.zeros_like(ex)], axis=1)
        placed = jnp.dot(dm_ref[...], r1_ref[...], preferred_element_type=jnp.float32) \
            * jnp.dot(ex, r2_ref[...], preferred_element_type=jnp.float32)
        oh = (dv_ref[...] == lax.broadcasted_iota(jnp.int32, (EB, NFOLD), 1).astype(jnp.float32))
        contrib = lax.dot_general(oh.astype(jnp.float32), placed,
                                  (((0,), (0,)), ((), ())),
                                  preferred_element_type=jnp.float32)

        @pl.when(i == 0)
        def _():
            den_ref[...] = contrib

        @pl.when(i > 0)
        def _():
            den_ref[...] += contrib

    return pl.pallas_call(
        body,
        grid=(NEB,),
        in_specs=[_edge_spec(D), _edge_spec(D, 0), _edge_spec(RBF), _edge_spec(1),
                  _edge_spec(16), _edge_spec(1),
                  _full((RBF, D)), _full((D, HEADS)), _full((16, D)),
                  _full((HEADS, D))],
        out_specs=[_edge_spec(16), pl.BlockSpec((NFOLD, D), lambda i: (0, 0))],
        out_shape=[jax.ShapeDtypeStruct((EP, 16), jnp.float32),
                   jax.ShapeDtypeStruct((NFOLD, D), jnp.float32)],
    )(qg, kvg, rbf_p, mask_p, dm16, ddiv, Wrbf, SEL, R1c, R2c)


def _tc_messages(ex, dtab, dm16, ddiv, kvg, rsh_p, env_p, Wg12, EXP8,
                 A1, B1, A2, B2, R1c, SS8):
    def body(ex_ref, dt_ref, dm_ref, dv_ref, v_ref, rsh_ref, env_ref,
             wg_ref, e8_ref, a1_ref, b1_ref, a2_ref, b2_ref, r1_ref, ss_ref,
             ms_ref, va_ref, vb_ref, vc_ref):
        oh = (dv_ref[...] == lax.broadcasted_iota(jnp.int32, (EB, NFOLD), 1).astype(jnp.float32))
        rowsel = jnp.dot(oh.astype(jnp.float32), dt_ref[...],
                         preferred_element_type=jnp.float32)
        msk = jnp.dot(dm_ref[...], r1_ref[...], preferred_element_type=jnp.float32)
        dsum = jnp.dot(rowsel * msk, ss_ref[...], preferred_element_type=jnp.float32)
        attn = ex_ref[:, :HEADS] / (dsum + 1e-16) * env_ref[...]
        ms = jnp.dot(attn, e8_ref[...], preferred_element_type=jnp.float32) * v_ref[...]
        g = jnp.dot(ms, wg_ref[...], preferred_element_type=jnp.float32)
        rsh = rsh_ref[...]
        v1 = jnp.dot(g[:, :N1], a1_ref[...], preferred_element_type=jnp.float32) \
            * jnp.dot(rsh[:, 0:3], b1_ref[...], preferred_element_type=jnp.float32)
        v2 = jnp.dot(g[:, N1:], a2_ref[...], preferred_element_type=jnp.float32) \
            * jnp.dot(rsh[:, 3:8], b2_ref[...], preferred_element_type=jnp.float32)
        ms_ref[...] = ms
        va_ref[...] = v1[:, :D]
        vb_ref[...] = jnp.concatenate([v1[:, D:], v2[:, :2 * D - N1 * 3]], axis=1)
        vc_ref[...] = jnp.concatenate(
            [v2[:, 2 * D - N1 * 3:],
             jnp.zeros((EB, 3 * D - N1 * 3 - N2 * 5), jnp.float32)], axis=1)

    return pl.pallas_call(
        body,
        grid=(NEB,),
        in_specs=[_edge_spec(16), _full((NFOLD, D)), _edge_spec(16), _edge_spec(1),
                  _edge_spec(D, 1), _edge_spec(8),
                  _edge_spec(1), _full((D, N1 + N2)), _full((HEADS, D)),
                  _full((N1, N1 * 3)), _full((3, N1 * 3)),
                  _full((N2, N2 * 5)), _full((5, N2 * 5)),
                  _full((16, D)), _full((D, HEADS))],
        out_specs=[_edge_spec(D), _edge_spec(D), _edge_spec(D), _edge_spec(D)],
        out_shape=[jax.ShapeDtypeStruct((EP, D), jnp.float32),
                   jax.ShapeDtypeStruct((EP, D), jnp.float32),
                   jax.ShapeDtypeStruct((EP, D), jnp.float32),
                   jax.ShapeDtypeStruct((EP, D), jnp.float32)],
    )(ex, dtab, dm16, ddiv, kvg, rsh_p, env_p, Wg12, EXP8, A1, B1, A2, B2, R1c, SS8)


def _tc_update(xs, xv, ms2, va2, vb2, vc2, root, consts):
    M1, M2, S3, S5, G1, G2, Wu1, bu1, Wu2, bu2, ln_g, ln_b, eq3, eq5 = consts

    def body(xs_ref, xv_ref, ms_ref, va_ref, vb_ref, vc_ref, rt_ref,
             m1_ref, m2_ref, s3_ref, s5_ref, g1_ref, g2_ref,
             wu1_ref, bu1_ref, wu2_ref, bu2_ref, lg_ref, lb_ref, e3_ref, e5_ref,
             xso_ref, xvo_ref):
        xs_in = xs_ref[...]
        xv_in = xv_ref[...]
        ms = ms_ref[...]
        va = va_ref[0] + va_ref[1]
        vb = vb_ref[0] + vb_ref[1]
        vc = vc_ref[0] + vc_ref[1]
        a1 = jnp.concatenate([va, vb[:, :N1 * 3 - D]], axis=1)
        a2 = jnp.concatenate([vb[:, N1 * 3 - D:], vc[:, :N2 * 5 - 2 * D + N1 * 3]], axis=1)
        new_s = ms[0] + ms[1] + rt_ref[...]
        xv1 = xv_in[:, :N1 * 3]
        xv2 = xv_in[:, N1 * 3:]
        nv1 = a1 + jnp.dot(xv1, m1_ref[...], preferred_element_type=jnp.float32)
        nv2 = a2 + jnp.dot(xv2, m2_ref[...], preferred_element_type=jnp.float32)
        n1 = jnp.sqrt(jnp.dot(nv1 * nv1, s3_ref[...], preferred_element_type=jnp.float32) + 1e-8)
        n2 = jnp.sqrt(jnp.dot(nv2 * nv2, s5_ref[...], preferred_element_type=jnp.float32) + 1e-8)
        h = jnp.concatenate([new_s, n1, n2], axis=1)
        h = jnp.dot(h, wu1_ref[...], preferred_element_type=jnp.float32) + bu1_ref[...]
        h = h * jax.nn.sigmoid(h)
        h = jnp.dot(h, wu2_ref[...], preferred_element_type=jnp.float32) + bu2_ref[...]
        s_out = h[:, :D]
        gates = jax.nn.sigmoid(h[:, D:])
        v1g = nv1 * jnp.dot(gates[:, :N1], g1_ref[...], preferred_element_type=jnp.float32)
        v2g = nv2 * jnp.dot(gates[:, N1:], g2_ref[...], preferred_element_type=jnp.float32)
        mu = jnp.mean(s_out, axis=-1, keepdims=True)
        dvi = s_out - mu
        var = jnp.mean(dvi * dvi, axis=-1, keepdims=True)
        xs_ln = dvi / jnp.sqrt(var + 1e-8) * lg_ref[...] + lb_ref[...]
        msq = (jnp.sum(v1g * v1g, axis=-1, keepdims=True)
               + jnp.sum(v2g * v2g, axis=-1, keepdims=True)) / (N1 + N2)
        inv = 1.0 / jnp.sqrt(msq + 1e-8)
        xv_ln = jnp.concatenate([v1g * inv * e3_ref[...], v2g * inv * e5_ref[...]], axis=1)
        xso_ref[...] = xs_ln + xs_in
        xvo_ref[...] = xv_ln + xv_in

    U = D + N1 + N2
    return pl.pallas_call(
        body,
        grid=(pl.cdiv(N, NB),),
        in_specs=[pl.BlockSpec((NB, D), lambda i: (i, 0)),
                  pl.BlockSpec((NB, VDIM), lambda i: (i, 0)),
                  pl.BlockSpec((NC, NB, D), lambda i: (0, i, 0)),
                  pl.BlockSpec((NC, NB, D), lambda i: (0, i, 0)),
                  pl.BlockSpec((NC, NB, D), lambda i: (0, i, 0)),
                  pl.BlockSpec((NC, NB, D), lambda i: (0, i, 0)),
                  pl.BlockSpec((NB, D), lambda i: (i, 0)),
                  _full((N1 * 3, N1 * 3)), _full((N2 * 5, N2 * 5)),
                  _full((N1 * 3, N1)), _full((N2 * 5, N2)),
                  _full((N1, N1 * 3)), _full((N2, N2 * 5)),
                  _full((U, 2 * U)), _full((1, 2 * U)), _full((2 * U, U)), _full((1, U)),
                  _full((1, D)), _full((1, D)), _full((1, N1 * 3)), _full((1, N2 * 5))],
        out_specs=[pl.BlockSpec((NB, D), lambda i: (i, 0)),
                   pl.BlockSpec((NB, VDIM), lambda i: (i, 0))],
        out_shape=[jax.ShapeDtypeStruct((N, D), jnp.float32),
                   jax.ShapeDtypeStruct((N, VDIM), jnp.float32)],
    )(xs, xv, ms2, va2, vb2, vc2, root,
      M1, M2, S3, S5, G1, G2, Wu1, bu1, Wu2, bu2, ln_g, ln_b, eq3, eq5)


def _tc_renv(rbf_p, env_p):
    def body(rbf_ref, env_ref, out_ref):
        re = rbf_ref[...] * env_ref[...]
        out_ref[...] = jnp.concatenate(
            [re, jnp.zeros((EB, D - RBF), jnp.float32)], axis=1)

    return pl.pallas_call(
        body,
        grid=(NEB,),
        in_specs=[_edge_spec(RBF), _edge_spec(1)],
        out_specs=_edge_spec(D),
        out_shape=jax.ShapeDtypeStruct((EP, D), jnp.float32),
    )(rbf_p, env_p)


def _tc_readout(xs, ne2, Wr, mlp, Woutp, boutp):
    (W1, b1), (W2, b2), (W3, b3) = mlp

    def body(xs_ref, ne_ref, wr_ref, w1_ref, b1_ref, w2_ref, b2_ref,
             w3_ref, b3_ref, wo_ref, bo_ref, out_ref):
        ne = ne_ref[...]
        node20 = ne[0, :, :RBF] + ne[1, :, :RBF]
        h = xs_ref[...] + jnp.dot(node20, wr_ref[...], preferred_element_type=jnp.float32)
        for w_ref, bias_ref in ((w1_ref, b1_ref), (w2_ref, b2_ref), (w3_ref, b3_ref)):
            h = jnp.dot(h, w_ref[...], preferred_element_type=jnp.float32) + bias_ref[...]
            h = h * jax.nn.sigmoid(h)
        out_ref[...] = jnp.dot(h, wo_ref[...], preferred_element_type=jnp.float32) + bo_ref[...]

    return pl.pallas_call(
        body,
        grid=(pl.cdiv(N, NB),),
        in_specs=[pl.BlockSpec((NB, D), lambda i: (i, 0)),
                  pl.BlockSpec((NC, NB, D), lambda i: (0, i, 0)),
                  _full((RBF, D)), _full((D, D)), _full((1, D)), _full((D, D)),
                  _full((1, D)), _full((D, D)), _full((1, D)), _full((D, 8)), _full((1, 8))],
        out_specs=pl.BlockSpec((NB, 8), lambda i: (i, 0)),
        out_shape=jax.ShapeDtypeStruct((N, 8), jnp.float32),
    )(xs, ne2, Wr, W1, b1, W2, b2, W3, b3, Woutp, boutp)


def _np_consts():
    SEL = np.zeros((D, HEADS), np.float32)
    EXP8 = np.zeros((HEADS, D), np.float32)
    for h in range(HEADS):
        SEL[h * HD:(h + 1) * HD, h] = 1.0
        EXP8[h, h * HD:(h + 1) * HD] = 1.0
    A1 = np.zeros((N1, N1 * 3), np.float32)
    B1 = np.zeros((3, N1 * 3), np.float32)
    for cc in range(N1):
        for dd in range(3):
            A1[cc, cc * 3 + dd] = 1.0
            B1[dd, cc * 3 + dd] = 1.0
    A2 = np.zeros((N2, N2 * 5), np.float32)
    B2 = np.zeros((5, N2 * 5), np.float32)
    for cc in range(N2):
        for dd in range(5):
            A2[cc, cc * 5 + dd] = 1.0
            B2[dd, cc * 5 + dd] = 1.0
    R1c = np.zeros((16, D), np.float32)
    R2c = np.zeros((HEADS, D), np.float32)
    SS8 = np.zeros((D, HEADS), np.float32)
    for ss in range(16):
        for h in range(HEADS):
            R1c[ss, ss * 8 + h] = 1.0
            R2c[h, ss * 8 + h] = 1.0
            SS8[ss * 8 + h, h] = 1.0
    return (jnp.asarray(SEL), jnp.asarray(EXP8), jnp.asarray(A1), jnp.asarray(B1),
            jnp.asarray(A2), jnp.asarray(B2), jnp.asarray(R1c), jnp.asarray(R2c),
            jnp.asarray(SS8))


def kernel(x_scalar, x_vector, edge_index, node_rbf, rsh, edge_index_0,
           atom_batch, envelop_para, params):
    SEL, EXP8, A1, B1, A2, B2, R1c, R2c, SS8 = _np_consts()
    pad = EP - E
    src = edge_index[0]
    dst = edge_index[1]
    dstp = jnp.pad(dst, (0, pad)).reshape(NW, KE, CH)
    srcp = jnp.pad(src, (0, pad)).reshape(NW, KE, CH)
    ei0p = jnp.pad(edge_index_0[1], (0, pad)).reshape(NW, KE, CH)
    abp = jnp.pad(atom_batch, (0, AP - N)).reshape(NW, KA, CH)
    mask_p = (jnp.arange(EP, dtype=jnp.int32) < E).astype(jnp.float32)[:, None]
    rbf_p = jnp.pad(node_rbf, ((0, pad), (0, 0)))
    dstf = jnp.pad(dst, (0, pad))
    dm16 = (jnp.mod(dstf, 16)[:, None] == jnp.arange(16)[None, :]).astype(jnp.float32)
    ddiv = (dstf // 16).astype(jnp.float32)[:, None]
    rsh_p = jnp.pad(rsh, ((0, pad), (0, 0)))
    env_p = jnp.pad(envelop_para, (0, pad))[:, None]

    xs, xv = x_scalar, x_vector
    for p in params['layers']:
        Wcat = jnp.concatenate([p['Wq'], p['Wk'], p['Wv'], p['Wroot']], axis=1)
        Wg12 = jnp.concatenate([p['Wg1'], p['Wg2']], axis=1)
        M1 = jnp.kron(p['Wv1'], jnp.eye(3, dtype=jnp.float32))
        M2 = jnp.kron(p['Wv2'], jnp.eye(5, dtype=jnp.float32))
        eq3 = jnp.repeat(p['eq_g'][:N1], 3)[None, :]
        eq5 = jnp.repeat(p['eq_g'][N1:], 5)[None, :]
        consts = (M1, M2, A1.T, A2.T, A1, A2,
                  p['Wu1'], p['bu1'][None, :], p['Wu2'], p['bu2'][None, :],
                  p['ln_g'][None, :], p['ln_b'][None, :], eq3, eq5)

        q, kv, root = _tc_project(xs, Wcat)
        qg = _sc_gather(q, dstp, k_chunks=KE, row_w=D)
        kvg = _sc_gather(kv, srcp, k_chunks=KE, row_w=2 * D)
        ex, dtab = _tc_logits(qg, kvg, rbf_p, mask_p, dm16, ddiv,
                              p['Wrbf'], SEL, R1c, R2c)
        ms, vma, vmb, vmc = _tc_messages(ex, dtab, dm16, ddiv, kvg, rsh_p,
                                         env_p, Wg12, EXP8, A1, B1, A2, B2,
                                         R1c, SS8)
        ms2 = _sc_scatter_add(ms, dstp, k_chunks=KE, row_w=D, n_rows=NPAD)
        va2 = _sc_scatter_add(vma, dstp, k_chunks=KE, row_w=D, n_rows=NPAD)
        vb2 = _sc_scatter_add(vmb, dstp, k_chunks=KE, row_w=D, n_rows=NPAD)
        vc2 = _sc_scatter_add(vmc, dstp, k_chunks=KE, row_w=D, n_rows=NPAD)
        xs, xv = _tc_update(xs, xv, ms2, va2, vb2, vc2, root, consts)

    ro = params['readout']
    renv = _tc_renv(rbf_p, env_p)
    ne2 = _sc_scatter_add(renv, ei0p, k_chunks=KE, row_w=D, n_rows=NPAD)
    mlp = [(w, b[None, :]) for (w, b) in ro['mlp']]
    Woutp = jnp.pad(ro['Wout'], ((0, 0), (0, 7)))
    boutp = jnp.pad(ro['bout'], (0, 7))[None, :]
    res = _tc_readout(xs, ne2, ro['Wr'], mlp, Woutp, boutp)
    resp = jnp.pad(res, ((0, AP - N), (0, D - 8)))
    out2 = _sc_scatter_add(resp, abp, k_chunks=KA, row_w=D, n_rows=128)
    return (out2[0, :NUM_GRAPHS, 0] + out2[1, :NUM_GRAPHS, 0])
